# baseline (speedup 1.0000x reference)
import functools

import jax
import jax.numpy as jnp
from jax import lax
from jax.experimental import pallas as pl
from jax.experimental.pallas import tpu as pltpu
from jax.experimental.pallas import tpu_sc as plsc

_N_EMBED = 1024
_EMBED_DIM = 256
_NUM_Q = 4
_BLK = 1152


def _argmin_body(r_ref, insq_ref, cbsq_ref, cbt_ref, idx_ref):
    r = r_ref[...]
    r16 = r.astype(jnp.bfloat16)
    mm = jnp.dot(r16, cbt_ref[...], preferred_element_type=jnp.float32)
    dist = insq_ref[...] + cbsq_ref[...] - 2.0 * mm
    m = jnp.min(dist, axis=1, keepdims=True)
    iota = jax.lax.broadcasted_iota(jnp.int32, (_BLK, _N_EMBED), 1)
    idx_ref[...] = jnp.min(jnp.where(dist == m, iota, _N_EMBED),
                           axis=1, keepdims=True)


def _final_body(z_ref, r3_ref, q3_ref, zq_ref, loss_ref):
    zb = z_ref[...]
    r4 = r3_ref[...] - q3_ref[...]
    quant = zb - r4
    zq_ref[...] = zb + (quant - zb)
    loss_ref[...] = jnp.broadcast_to(jnp.sum((zb - quant) ** 2), (1, 8, 128))


def _sc_gather(table, idx, n_tok):
    info = plsc.get_sparse_core_info()
    nw = info.num_cores * info.num_subcores
    b_per_w = n_tok // nw
    mesh = plsc.VectorSubcoreMesh(core_axis_name="c", subcore_axis_name="s")

    @functools.partial(
        pl.kernel, mesh=mesh,
        out_type=jax.ShapeDtypeStruct((n_tok, _EMBED_DIM), jnp.float32),
        scratch_types=[
            pltpu.VMEM((b_per_w,), jnp.int32),
            pltpu.VMEM((b_per_w, _EMBED_DIM), jnp.float32),
            pltpu.SemaphoreType.DMA,
        ],
    )
    def gk(table_hbm, idx_hbm, out_hbm, idx_v, rows_v, sem):
        wid = lax.axis_index("s") * info.num_cores + lax.axis_index("c")
        base = wid * b_per_w
        pltpu.sync_copy(idx_hbm.at[pl.ds(base, b_per_w)], idx_v)
        pltpu.async_copy(table_hbm.at[idx_v], rows_v, sem).wait()
        pltpu.sync_copy(rows_v, out_hbm.at[pl.ds(base, b_per_w)])

    return gk(table, idx)


def _tok_spec():
    return pl.BlockSpec((_BLK, _EMBED_DIM), lambda i: (i, 0))


def _col_spec():
    return pl.BlockSpec((_BLK, 1), lambda i: (i, 0))


_PARAMS = pltpu.CompilerParams(dimension_semantics=("parallel",))


def _half_pipeline(z_half, codebooks, cbt16, cb_sqs):
    n_tok = z_half.shape[0]
    grid = (n_tok // _BLK,)
    f32 = jnp.float32
    idx_shape = jax.ShapeDtypeStruct((n_tok, 1), jnp.int32)

    r = z_half
    codes_cols = []
    for l in range(_NUM_Q):
        in_sq = jnp.sum(r * r, axis=1, keepdims=True)
        idx = pl.pallas_call(
            _argmin_body, grid=grid,
            in_specs=[_tok_spec(), _col_spec(),
                      pl.BlockSpec((1, _N_EMBED), lambda i: (0, 0)),
                      pl.BlockSpec((_EMBED_DIM, _N_EMBED), lambda i: (0, 0))],
            out_specs=_col_spec(),
            out_shape=idx_shape,
            compiler_params=_PARAMS,
        )(r, in_sq, cb_sqs[l], cbt16[l])
        codes_cols.append(idx)
        q = _sc_gather(codebooks[l], idx.reshape(n_tok), n_tok)
        if l < _NUM_Q - 1:
            r = r - q
        else:
            z_q, loss_parts = pl.pallas_call(
                _final_body, grid=grid,
                in_specs=[_tok_spec(), _tok_spec(), _tok_spec()],
                out_specs=[_tok_spec(),
                           pl.BlockSpec((1, 8, 128), lambda i: (i, 0, 0))],
                out_shape=[jax.ShapeDtypeStruct((n_tok, _EMBED_DIM), f32),
                           jax.ShapeDtypeStruct((grid[0], 8, 128), f32)],
                compiler_params=_PARAMS,
            )(z_half, r, q)
    codes = jnp.concatenate(codes_cols, axis=1)
    loss_sum = jnp.sum(loss_parts[:, 0, 0])
    return z_q, codes, loss_sum


def kernel(z, codebooks):
    n_tok = z.shape[0] * z.shape[1]
    z_flat = z.reshape(n_tok, _EMBED_DIM)

    cbt16 = codebooks.transpose(0, 2, 1).astype(jnp.bfloat16)
    cb_sqs = [jnp.sum(codebooks[l] * codebooks[l], axis=1)[None, :]
              for l in range(_NUM_Q)]

    half = n_tok // 2
    za, zb_ = z_flat[:half], z_flat[half:]
    zq_a, codes_a, loss_a = _half_pipeline(za, codebooks, cbt16, cb_sqs)
    zq_b, codes_b, loss_b = _half_pipeline(zb_, codebooks, cbt16, cb_sqs)

    z_q = jnp.concatenate([zq_a, zq_b], axis=0).reshape(z.shape)
    codes = jnp.concatenate([codes_a, codes_b], axis=0).reshape(
        z.shape[0], z.shape[1], _NUM_Q)
    commit_loss = (loss_a + loss_b) / (n_tok * _EMBED_DIM)
    return z_q, codes, commit_loss


# speedup vs baseline: 1.1427x; 1.1427x over previous
import functools

import jax
import jax.numpy as jnp
from jax import lax
from jax.experimental import pallas as pl
from jax.experimental.pallas import tpu as pltpu
from jax.experimental.pallas import tpu_sc as plsc

_N_EMBED = 1024
_EMBED_DIM = 256
_NUM_Q = 4
_BLK = 1152


def _distances(r, insq_ref, cbsq_ref, cbt_ref):
    r16 = r.astype(jnp.bfloat16)
    mm = jnp.dot(r16, cbt_ref[...], preferred_element_type=jnp.float32)
    dist = insq_ref[...] + cbsq_ref[...] - 2.0 * mm
    m = jnp.min(dist, axis=1, keepdims=True)
    iota = jax.lax.broadcasted_iota(jnp.int32, (_BLK, _N_EMBED), 1)
    idx = jnp.min(jnp.where(dist == m, iota, _N_EMBED),
                  axis=1, keepdims=True)
    return idx, iota


def _argmin_body(r_ref, insq_ref, cbsq_ref, cbt_ref, idx_ref):
    idx, _ = _distances(r_ref[...], insq_ref, cbsq_ref, cbt_ref)
    idx_ref[...] = idx


def _last_body(r_ref, insq_ref, cbsq_ref, cbt_ref, w3_ref, z_ref,
               zq_ref, idx_ref, loss_ref):
    r = r_ref[...]
    idx, iota = _distances(r, insq_ref, cbsq_ref, cbt_ref)
    oh = (iota == idx).astype(jnp.bfloat16)
    oh3 = jnp.concatenate([oh, oh, oh], axis=1)
    q = jnp.dot(oh3, w3_ref[...], preferred_element_type=jnp.float32)
    r4 = r - q
    zb = z_ref[...]
    quant = zb - r4
    zq_ref[...] = zb + (quant - zb)
    idx_ref[...] = idx
    loss_ref[...] = jnp.broadcast_to(jnp.sum((zb - quant) ** 2), (1, 8, 128))


def _sc_gather(table, idx, n_tok):
    info = plsc.get_sparse_core_info()
    nw = info.num_cores * info.num_subcores
    b_per_w = n_tok // nw
    mesh = plsc.VectorSubcoreMesh(core_axis_name="c", subcore_axis_name="s")

    @functools.partial(
        pl.kernel, mesh=mesh,
        out_type=jax.ShapeDtypeStruct((n_tok, _EMBED_DIM), jnp.float32),
        scratch_types=[
            pltpu.VMEM((b_per_w,), jnp.int32),
            pltpu.VMEM((b_per_w, _EMBED_DIM), jnp.float32),
            pltpu.SemaphoreType.DMA,
        ],
    )
    def gk(table_hbm, idx_hbm, out_hbm, idx_v, rows_v, sem):
        wid = lax.axis_index("s") * info.num_cores + lax.axis_index("c")
        base = wid * b_per_w
        pltpu.sync_copy(idx_hbm.at[pl.ds(base, b_per_w)], idx_v)
        pltpu.async_copy(table_hbm.at[idx_v], rows_v, sem).wait()
        pltpu.sync_copy(rows_v, out_hbm.at[pl.ds(base, b_per_w)])

    return gk(table, idx)


def _tok_spec():
    return pl.BlockSpec((_BLK, _EMBED_DIM), lambda i: (i, 0))


def _col_spec():
    return pl.BlockSpec((_BLK, 1), lambda i: (i, 0))


_PARAMS = pltpu.CompilerParams(dimension_semantics=("parallel",))


def kernel(z, codebooks):
    n_tok = z.shape[0] * z.shape[1]
    grid = (n_tok // _BLK,)
    z_flat = z.reshape(n_tok, _EMBED_DIM)
    f32 = jnp.float32

    cbt16 = codebooks.transpose(0, 2, 1).astype(jnp.bfloat16)
    mask = jnp.uint32(0xFFFF0000)
    cb_last = codebooks[_NUM_Q - 1]
    u = jax.lax.bitcast_convert_type(cb_last, jnp.uint32)
    hi32 = jax.lax.bitcast_convert_type(u & mask, f32)
    r1 = cb_last - hi32
    u1 = jax.lax.bitcast_convert_type(r1, jnp.uint32)
    mid32 = jax.lax.bitcast_convert_type(u1 & mask, f32)
    lo32 = r1 - mid32
    w3_last = jnp.concatenate([hi32.astype(jnp.bfloat16),
                               mid32.astype(jnp.bfloat16),
                               lo32.astype(jnp.bfloat16)], axis=0)

    idx_shape = jax.ShapeDtypeStruct((n_tok, 1), jnp.int32)

    r = z_flat
    codes_cols = []
    for l in range(_NUM_Q):
        cb_l = codebooks[l]
        cb_sq = jnp.sum(cb_l * cb_l, axis=1)[None, :]
        in_sq = jnp.sum(r * r, axis=1, keepdims=True)
        if l < _NUM_Q - 1:
            idx = pl.pallas_call(
                _argmin_body, grid=grid,
                in_specs=[_tok_spec(), _col_spec(),
                          pl.BlockSpec((1, _N_EMBED), lambda i: (0, 0)),
                          pl.BlockSpec((_EMBED_DIM, _N_EMBED),
                                       lambda i: (0, 0))],
                out_specs=_col_spec(),
                out_shape=idx_shape,
                compiler_params=_PARAMS,
            )(r, in_sq, cb_sq, cbt16[l])
            q = _sc_gather(cb_l, idx.reshape(n_tok), n_tok)
            r = r - q
        else:
            z_q, idx, loss_parts = pl.pallas_call(
                _last_body, grid=grid,
                in_specs=[_tok_spec(), _col_spec(),
                          pl.BlockSpec((1, _N_EMBED), lambda i: (0, 0)),
                          pl.BlockSpec((_EMBED_DIM, _N_EMBED),
                                       lambda i: (0, 0)),
                          pl.BlockSpec((3 * _N_EMBED, _EMBED_DIM),
                                       lambda i: (0, 0)),
                          _tok_spec()],
                out_specs=[_tok_spec(), _col_spec(),
                           pl.BlockSpec((1, 8, 128), lambda i: (i, 0, 0))],
                out_shape=[jax.ShapeDtypeStruct((n_tok, _EMBED_DIM), f32),
                           idx_shape,
                           jax.ShapeDtypeStruct((grid[0], 8, 128), f32)],
                compiler_params=_PARAMS,
            )(r, in_sq, cb_sq, cbt16[l], w3_last, z_flat)
        codes_cols.append(idx)

    codes = jnp.concatenate(codes_cols, axis=1).reshape(
        z.shape[0], z.shape[1], _NUM_Q)
    z_q = z_q.reshape(z.shape)
    commit_loss = jnp.sum(loss_parts[:, 0, 0]) / (n_tok * _EMBED_DIM)
    return z_q, codes, commit_loss


# speedup vs baseline: 1.1658x; 1.0202x over previous
import functools

import jax
import jax.numpy as jnp
from jax import lax
from jax.experimental import pallas as pl
from jax.experimental.pallas import tpu as pltpu
from jax.experimental.pallas import tpu_sc as plsc

_N_EMBED = 1024
_EMBED_DIM = 256
_NUM_Q = 4
_BLK = 1152
_BLK_A = 2304


def _distances(r, insq_ref, cbsq_ref, cbt_ref, blk):
    r16 = r.astype(jnp.bfloat16)
    mm = jnp.dot(r16, cbt_ref[...], preferred_element_type=jnp.float32)
    dist = insq_ref[...] + cbsq_ref[...] - 2.0 * mm
    m = jnp.min(dist, axis=1, keepdims=True)
    iota = jax.lax.broadcasted_iota(jnp.int32, (blk, _N_EMBED), 1)
    idx = jnp.min(jnp.where(dist == m, iota, _N_EMBED),
                  axis=1, keepdims=True)
    return idx, iota


def _argmin_body(r_ref, insq_ref, cbsq_ref, cbt_ref, idx_ref):
    idx, _ = _distances(r_ref[...], insq_ref, cbsq_ref, cbt_ref, _BLK_A)
    idx_ref[...] = idx


def _last_body(r_ref, insq_ref, cbsq_ref, cbt_ref, w3_ref, z_ref,
               zq_ref, idx_ref, loss_ref):
    r = r_ref[...]
    idx, iota = _distances(r, insq_ref, cbsq_ref, cbt_ref, _BLK)
    oh = (iota == idx).astype(jnp.bfloat16)
    oh3 = jnp.concatenate([oh, oh, oh], axis=1)
    q = jnp.dot(oh3, w3_ref[...], preferred_element_type=jnp.float32)
    r4 = r - q
    zb = z_ref[...]
    quant = zb - r4
    zq_ref[...] = zb + (quant - zb)
    idx_ref[...] = idx
    loss_ref[...] = jnp.broadcast_to(jnp.sum((zb - quant) ** 2), (1, 8, 128))


def _sc_gather(table, idx, n_tok):
    info = plsc.get_sparse_core_info()
    nw = info.num_cores * info.num_subcores
    b_per_w = n_tok // nw
    mesh = plsc.VectorSubcoreMesh(core_axis_name="c", subcore_axis_name="s")

    @functools.partial(
        pl.kernel, mesh=mesh,
        out_type=jax.ShapeDtypeStruct((n_tok, _EMBED_DIM), jnp.float32),
        scratch_types=[
            pltpu.VMEM((b_per_w,), jnp.int32),
            pltpu.VMEM((b_per_w, _EMBED_DIM), jnp.float32),
            pltpu.SemaphoreType.DMA,
        ],
    )
    def gk(table_hbm, idx_hbm, out_hbm, idx_v, rows_v, sem):
        wid = lax.axis_index("s") * info.num_cores + lax.axis_index("c")
        base = wid * b_per_w
        pltpu.sync_copy(idx_hbm.at[pl.ds(base, b_per_w)], idx_v)
        pltpu.async_copy(table_hbm.at[idx_v], rows_v, sem).wait()
        pltpu.sync_copy(rows_v, out_hbm.at[pl.ds(base, b_per_w)])

    return gk(table, idx)


def _tok_spec():
    return pl.BlockSpec((_BLK, _EMBED_DIM), lambda i: (i, 0))


def _col_spec():
    return pl.BlockSpec((_BLK, 1), lambda i: (i, 0))


_PARAMS = pltpu.CompilerParams(dimension_semantics=("parallel",))


def kernel(z, codebooks):
    n_tok = z.shape[0] * z.shape[1]
    grid = (n_tok // _BLK,)
    z_flat = z.reshape(n_tok, _EMBED_DIM)
    f32 = jnp.float32

    cbt16 = codebooks.transpose(0, 2, 1).astype(jnp.bfloat16)
    mask = jnp.uint32(0xFFFF0000)
    cb_last = codebooks[_NUM_Q - 1]
    u = jax.lax.bitcast_convert_type(cb_last, jnp.uint32)
    hi32 = jax.lax.bitcast_convert_type(u & mask, f32)
    r1 = cb_last - hi32
    u1 = jax.lax.bitcast_convert_type(r1, jnp.uint32)
    mid32 = jax.lax.bitcast_convert_type(u1 & mask, f32)
    lo32 = r1 - mid32
    w3_last = jnp.concatenate([hi32.astype(jnp.bfloat16),
                               mid32.astype(jnp.bfloat16),
                               lo32.astype(jnp.bfloat16)], axis=0)

    idx_shape = jax.ShapeDtypeStruct((n_tok, 1), jnp.int32)

    r = z_flat
    codes_cols = []
    for l in range(_NUM_Q):
        cb_l = codebooks[l]
        cb_sq = jnp.sum(cb_l * cb_l, axis=1)[None, :]
        in_sq = jnp.sum(r * r, axis=1, keepdims=True)
        if l < _NUM_Q - 1:
            idx = pl.pallas_call(
                _argmin_body, grid=(n_tok // _BLK_A,),
                in_specs=[pl.BlockSpec((_BLK_A, _EMBED_DIM),
                                       lambda i: (i, 0)),
                          pl.BlockSpec((_BLK_A, 1), lambda i: (i, 0)),
                          pl.BlockSpec((1, _N_EMBED), lambda i: (0, 0)),
                          pl.BlockSpec((_EMBED_DIM, _N_EMBED),
                                       lambda i: (0, 0))],
                out_specs=pl.BlockSpec((_BLK_A, 1), lambda i: (i, 0)),
                out_shape=idx_shape,
                compiler_params=_PARAMS,
            )(r, in_sq, cb_sq, cbt16[l])
            q = _sc_gather(cb_l, idx.reshape(n_tok), n_tok)
            r = r - q
        else:
            z_q, idx, loss_parts = pl.pallas_call(
                _last_body, grid=grid,
                in_specs=[_tok_spec(), _col_spec(),
                          pl.BlockSpec((1, _N_EMBED), lambda i: (0, 0)),
                          pl.BlockSpec((_EMBED_DIM, _N_EMBED),
                                       lambda i: (0, 0)),
                          pl.BlockSpec((3 * _N_EMBED, _EMBED_DIM),
                                       lambda i: (0, 0)),
                          _tok_spec()],
                out_specs=[_tok_spec(), _col_spec(),
                           pl.BlockSpec((1, 8, 128), lambda i: (i, 0, 0))],
                out_shape=[jax.ShapeDtypeStruct((n_tok, _EMBED_DIM), f32),
                           idx_shape,
                           jax.ShapeDtypeStruct((grid[0], 8, 128), f32)],
                compiler_params=_PARAMS,
            )(r, in_sq, cb_sq, cbt16[l], w3_last, z_flat)
        codes_cols.append(idx)

    codes = jnp.concatenate(codes_cols, axis=1).reshape(
        z.shape[0], z.shape[1], _NUM_Q)
    z_q = z_q.reshape(z.shape)
    commit_loss = jnp.sum(loss_parts[:, 0, 0]) / (n_tok * _EMBED_DIM)
    return z_q, codes, commit_loss
